# baseline (speedup 1.0000x reference)
import jax
import jax.numpy as jnp
from jax.experimental import pallas as pl
from jax.experimental.pallas import tpu as pltpu

BLOCK_M = 1024
N_STREAMS = 2


def _router_kernel(xa_ref, xb_ref, wt_ref, b_ref, o_ref):
    wt = wt_ref[...]
    b = b_ref[...]

    def _softmax_block(x_blk):
        logits = jnp.dot(x_blk.astype(jnp.bfloat16), wt,
                         preferred_element_type=jnp.float32) + b
        m = jnp.max(logits, axis=-1, keepdims=True)
        e = jnp.exp(logits - m)
        return e / jnp.sum(e, axis=-1, keepdims=True)

    o_ref[0:BLOCK_M, :] = _softmax_block(xa_ref[...])
    o_ref[BLOCK_M:2 * BLOCK_M, :] = _softmax_block(xb_ref[...])


def kernel(x, W, b):
    n_tokens, embed_dim = x.shape
    n_experts = W.shape[0]
    wt = W.T.astype(jnp.bfloat16)
    b2 = b.reshape(1, n_experts)
    rows_per_step = BLOCK_M * N_STREAMS
    grid = (n_tokens // rows_per_step,)
    return pl.pallas_call(
        _router_kernel,
        grid=grid,
        in_specs=[
            pl.BlockSpec((BLOCK_M, embed_dim), lambda i: (2 * i, 0)),
            pl.BlockSpec((BLOCK_M, embed_dim), lambda i: (2 * i + 1, 0)),
            pl.BlockSpec((embed_dim, n_experts), lambda i: (0, 0)),
            pl.BlockSpec((1, n_experts), lambda i: (0, 0)),
        ],
        out_specs=pl.BlockSpec((rows_per_step, n_experts), lambda i: (i, 0)),
        out_shape=jax.ShapeDtypeStruct((n_tokens, n_experts), jnp.float32),
        compiler_params=pltpu.CompilerParams(
            dimension_semantics=("arbitrary",),
        ),
    )(x, x, wt, b2)


# speedup vs baseline: 1.0536x; 1.0536x over previous
import jax
import jax.numpy as jnp
from jax.experimental import pallas as pl
from jax.experimental.pallas import tpu as pltpu

BLOCK_M = 1024


def _router_kernel(x_ref, w_ref, b_ref, o_ref):
    w = w_ref[...].astype(jnp.bfloat16)
    logits = jax.lax.dot_general(
        x_ref[...].astype(jnp.bfloat16), w,
        dimension_numbers=(((1,), (1,)), ((), ())),
        preferred_element_type=jnp.float32)
    logits = logits + b_ref[...]
    m = jnp.max(logits, axis=-1, keepdims=True)
    e = jnp.exp(logits - m)
    o_ref[...] = e / jnp.sum(e, axis=-1, keepdims=True)


def kernel(x, W, b):
    n_tokens, embed_dim = x.shape
    n_experts = W.shape[0]
    b2 = b.reshape(1, n_experts)
    grid = (n_tokens // BLOCK_M,)
    return pl.pallas_call(
        _router_kernel,
        grid=grid,
        in_specs=[
            pl.BlockSpec((BLOCK_M, embed_dim), lambda i: (i, 0)),
            pl.BlockSpec((n_experts, embed_dim), lambda i: (0, 0)),
            pl.BlockSpec((1, n_experts), lambda i: (0, 0)),
        ],
        out_specs=pl.BlockSpec((BLOCK_M, n_experts), lambda i: (i, 0)),
        out_shape=jax.ShapeDtypeStruct((n_tokens, n_experts), jnp.float32),
        compiler_params=pltpu.CompilerParams(
            dimension_semantics=("arbitrary",),
        ),
    )(x, W, b2)
